# baseline (speedup 1.0000x reference)
import functools

import jax
import jax.numpy as jnp
from jax import lax
from jax.experimental import pallas as pl
from jax.experimental.pallas import tpu as pltpu
from jax.experimental.pallas import tpu_sc as plsc

N = 10000
E = 320000
D = 128
DE = 16
G = 16
DU = 32

NC = 2
NS = 16
NW = NC * NS
NCHUNK = 2
EC = E // NCHUNK
EW = EC // NW
C = 40
K = 25
S = K * C
NSUP = EW // S
NP = 10240
RPS = NP // NS

BE = 40000
BN = 2048


def _eap_body(ea_ref, wbig_ref, be_ref, out_ref):
    out_ref[...] = (
        jnp.dot(ea_ref[...], wbig_ref[...], preferred_element_type=jnp.float32)
        + be_ref[...]
    )


def _proj_body(x_ref, batch_ref, u_ref, we_ref, asrc_ref, xd_ref):
    x = x_ref[...]
    xs = jnp.dot(x, we_ref[0:D, :], preferred_element_type=jnp.float32)
    xd = jnp.dot(x, we_ref[D:2 * D, :], preferred_element_type=jnp.float32)
    uweu = jnp.dot(u_ref[...], we_ref[2 * D + DE:2 * D + DE + DU, :],
                   preferred_element_type=jnp.float32)
    oh = (batch_ref[...] == lax.broadcasted_iota(jnp.int32, (N, G), 1)
          ).astype(jnp.float32)
    asrc_ref[...] = xs + jnp.dot(oh, uweu, preferred_element_type=jnp.float32)
    xd_ref[...] = xd


def _edge_sc_body(asrc_hbm, xd_hbm, eap_hbm, sidx_hbm, didx_hbm, zeros_hbm,
                  ea2_hbm, aggp_hbm,
                  sidx_v, didx_v, abuf, bbuf, obuf, agg_sh,
                  sem_i, sem_e, sem_g, sem_o, sem_s):
    c = lax.axis_index("c")
    s = lax.axis_index("s")
    wid = c * NS + s

    pltpu.sync_copy(zeros_hbm.at[pl.ds(s * RPS, RPS), :],
                    agg_sh.at[pl.ds(s * RPS, RPS), :])
    plsc.subcore_barrier()

    rbase0 = wid * (EW // C)

    def sup_body(i, carry):
        rbase = rbase0 + i * K
        ebase = (rbase0 + i * K) * C
        ci = pltpu.async_copy(sidx_hbm.at[pl.ds(rbase, K), :], sidx_v, sem_i)
        cj = pltpu.async_copy(didx_hbm.at[pl.ds(rbase, K), :], didx_v, sem_i)
        ce = pltpu.async_copy(eap_hbm.at[pl.ds(ebase // 8, S // 8), :],
                              obuf, sem_e)
        ci.wait()
        cj.wait()

        def gfire(j, carry2):
            pltpu.async_copy(asrc_hbm.at[sidx_v.at[j]],
                             abuf.at[pl.ds(j * C, C), :], sem_g)
            pltpu.async_copy(xd_hbm.at[didx_v.at[j]],
                             bbuf.at[pl.ds(j * C, C), :], sem_g)
            return carry2

        lax.fori_loop(0, K, gfire, 0)
        pltpu.make_async_copy(asrc_hbm.at[pl.ds(0, S), :], abuf, sem_g).wait()
        pltpu.make_async_copy(asrc_hbm.at[pl.ds(0, S), :], bbuf, sem_g).wait()
        ce.wait()

        def row(rr, _):
            for k in range(8):
                j = rr * 8 + k
                v = abuf[j, :] + bbuf[j, :] + obuf[rr, k * 16:(k + 1) * 16]
                r = jnp.maximum(v, 0.0)
                abuf[j, :] = r
                obuf[rr, k * 16:(k + 1) * 16] = r
            return 0

        lax.fori_loop(0, S // 8, row, 0, unroll=2)
        co = pltpu.async_copy(obuf, ea2_hbm.at[pl.ds(ebase // 8, S // 8), :],
                              sem_o)

        def sfire(j, carry2):
            pltpu.async_copy(abuf.at[pl.ds(j * C, C), :],
                             agg_sh.at[didx_v.at[j]], sem_s, add=True)
            return carry2

        lax.fori_loop(0, K, sfire, 0)
        pltpu.make_async_copy(asrc_hbm.at[pl.ds(0, S), :], abuf, sem_s).wait()
        co.wait()
        return carry

    lax.fori_loop(0, NSUP, sup_body, 0)

    plsc.subcore_barrier()
    pltpu.sync_copy(agg_sh.at[pl.ds(s * RPS, RPS), :],
                    abuf.at[pl.ds(0, RPS), :])

    def pack(rr, _):
        for k in range(8):
            obuf[rr, k * 16:(k + 1) * 16] = abuf[rr * 8 + k, :]
        return 0

    lax.fori_loop(0, RPS // 8, pack, 0, unroll=2)
    pltpu.sync_copy(obuf.at[pl.ds(0, RPS // 8), :],
                    aggp_hbm.at[c, pl.ds(s * (RPS // 8), RPS // 8), :])


@functools.cache
def _edge_sc():
    return pl.kernel(
        _edge_sc_body,
        out_type=[
            jax.ShapeDtypeStruct((EC // 8, 128), jnp.float32),
            jax.ShapeDtypeStruct((NC, NP * DE // 128, 128), jnp.float32),
        ],
        mesh=plsc.VectorSubcoreMesh(core_axis_name="c", subcore_axis_name="s",
                                    num_cores=NC, num_subcores=NS),
        compiler_params=pltpu.CompilerParams(use_tc_tiling_on_sc=False),
        scratch_types=[
            pltpu.VMEM((K, C), jnp.int32),
            pltpu.VMEM((K, C), jnp.int32),
            pltpu.VMEM((S, DE), jnp.float32),
            pltpu.VMEM((S, DE), jnp.float32),
            pltpu.VMEM((S // 8, 128), jnp.float32),
            pltpu.VMEM_SHARED((NP, DE), jnp.float32),
            pltpu.SemaphoreType.DMA,
            pltpu.SemaphoreType.DMA,
            pltpu.SemaphoreType.DMA,
            pltpu.SemaphoreType.DMA,
            pltpu.SemaphoreType.DMA,
        ],
    )


def _node_body(x_ref, aggp_ref, batch_ref, pp_ref, u_ref, wn_ref, bn_ref,
               wg_ref, bg_ref, x2_ref, u2_ref, accx_ref, accp_ref, accc_ref):
    i = pl.program_id(0)
    x = x_ref[...]
    agg = aggp_ref[...]
    oh = (batch_ref[...] == lax.broadcasted_iota(jnp.int32, (BN, G), 1)
          ).astype(jnp.float32)
    uwnu = jnp.dot(u_ref[...], wn_ref[D + DE:D + DE + DU, :],
                   preferred_element_type=jnp.float32)
    h = (jnp.dot(x, wn_ref[0:D, :], preferred_element_type=jnp.float32)
         + jnp.dot(agg, wn_ref[D:D + DE, :], preferred_element_type=jnp.float32)
         + jnp.dot(oh, uwnu, preferred_element_type=jnp.float32)
         + bn_ref[...])
    x2 = jnp.maximum(h, 0.0)
    x2_ref[...] = x2

    @pl.when(i == 0)
    def _():
        accx_ref[...] = jnp.zeros_like(accx_ref)
        accp_ref[...] = jnp.zeros_like(accp_ref)
        accc_ref[...] = jnp.zeros_like(accc_ref)

    dn = (((0,), (0,)), ((), ()))
    accx_ref[...] += lax.dot_general(oh, x2, dn,
                                     preferred_element_type=jnp.float32)
    accp_ref[...] += lax.dot_general(oh, pp_ref[...], dn,
                                     preferred_element_type=jnp.float32)
    accc_ref[...] += lax.dot_general(oh, jnp.ones((BN, D), jnp.float32), dn,
                                     preferred_element_type=jnp.float32)

    @pl.when(i == pl.num_programs(0) - 1)
    def _():
        cnt = jnp.maximum(accc_ref[...], 1.0)
        px = accx_ref[...] / cnt
        ppool = accp_ref[...] / cnt[:, 0:2]
        g = (jnp.dot(px, wg_ref[0:D, :], preferred_element_type=jnp.float32)
             + jnp.dot(ppool, wg_ref[D:D + 2, :],
                       preferred_element_type=jnp.float32)
             + jnp.dot(u_ref[...], wg_ref[D + 2:D + 2 + DU, :],
                       preferred_element_type=jnp.float32)
             + bg_ref[...])
        u2_ref[...] = jnp.maximum(g, 0.0)


def kernel(x, edge_index, edge_attr, u, batch, polar_pos, We, be, Wn, bn,
           Wg, bg):
    src = edge_index[0].astype(jnp.int32).reshape(E // C, C)
    dst = edge_index[1].astype(jnp.int32).reshape(E // C, C)
    batch2d = batch.astype(jnp.int32).reshape(N, 1)
    bn2 = bn.reshape(1, D)
    bg2 = bg.reshape(1, DU)
    zeros = jnp.zeros((NP, DE), jnp.float32)

    wbig = jnp.kron(jnp.eye(8, dtype=jnp.float32), We[2 * D:2 * D + DE, :])
    be128 = jnp.tile(be, 8).reshape(1, 128)

    a_src, xd = pl.pallas_call(
        _proj_body,
        in_specs=[
            pl.BlockSpec((N, D), lambda: (0, 0)),
            pl.BlockSpec((N, 1), lambda: (0, 0)),
            pl.BlockSpec((G, DU), lambda: (0, 0)),
            pl.BlockSpec((2 * D + DE + DU, DE), lambda: (0, 0)),
        ],
        out_specs=[
            pl.BlockSpec((N, DE), lambda: (0, 0)),
            pl.BlockSpec((N, DE), lambda: (0, 0)),
        ],
        out_shape=[
            jax.ShapeDtypeStruct((N, DE), jnp.float32),
            jax.ShapeDtypeStruct((N, DE), jnp.float32),
        ],
    )(x, batch2d, u, We)

    rows = EC // C
    ea2p_parts = []
    aggp_parts = []
    for ci in range(NCHUNK):
        ea128_c = lax.slice(edge_attr, (ci * EC, 0),
                            ((ci + 1) * EC, DE)).reshape(EC // 8, 128)
        eap_c = pl.pallas_call(
            _eap_body,
            grid=(EC // BE,),
            in_specs=[
                pl.BlockSpec((BE // 8, 128), lambda i: (i, 0)),
                pl.BlockSpec((128, 128), lambda i: (0, 0)),
                pl.BlockSpec((1, 128), lambda i: (0, 0)),
            ],
            out_specs=pl.BlockSpec((BE // 8, 128), lambda i: (i, 0)),
            out_shape=jax.ShapeDtypeStruct((EC // 8, 128), jnp.float32),
        )(ea128_c, wbig, be128)
        src_c = lax.slice(src, (ci * rows, 0), ((ci + 1) * rows, C))
        dst_c = lax.slice(dst, (ci * rows, 0), ((ci + 1) * rows, C))
        ea2p_c, aggp_c = _edge_sc()(a_src, xd, eap_c, src_c, dst_c, zeros)
        ea2p_parts.append(ea2p_c)
        aggp_parts.append(aggp_c)

    ea2 = jnp.concatenate(ea2p_parts, axis=0).reshape(E, DE)
    agg2 = sum(p[0] + p[1] for p in aggp_parts).reshape(NP, DE)

    xp = jnp.pad(x, ((0, NP - N), (0, 0)))
    batchp = jnp.pad(batch2d, ((0, NP - N), (0, 0)), constant_values=G)
    ppp = jnp.pad(polar_pos, ((0, NP - N), (0, 0)))

    x2p, u2 = pl.pallas_call(
        _node_body,
        grid=(NP // BN,),
        in_specs=[
            pl.BlockSpec((BN, D), lambda i: (i, 0)),
            pl.BlockSpec((BN, DE), lambda i: (i, 0)),
            pl.BlockSpec((BN, 1), lambda i: (i, 0)),
            pl.BlockSpec((BN, 2), lambda i: (i, 0)),
            pl.BlockSpec((G, DU), lambda i: (0, 0)),
            pl.BlockSpec((D + DE + DU, D), lambda i: (0, 0)),
            pl.BlockSpec((1, D), lambda i: (0, 0)),
            pl.BlockSpec((D + 2 + DU, DU), lambda i: (0, 0)),
            pl.BlockSpec((1, DU), lambda i: (0, 0)),
        ],
        out_specs=[
            pl.BlockSpec((BN, D), lambda i: (i, 0)),
            pl.BlockSpec((G, DU), lambda i: (0, 0)),
        ],
        out_shape=[
            jax.ShapeDtypeStruct((NP, D), jnp.float32),
            jax.ShapeDtypeStruct((G, DU), jnp.float32),
        ],
        scratch_shapes=[
            pltpu.VMEM((G, D), jnp.float32),
            pltpu.VMEM((G, 2), jnp.float32),
            pltpu.VMEM((G, D), jnp.float32),
        ],
    )(xp, agg2, batchp, ppp, u, Wn, bn2, Wg, bg2)

    return (x2p[:N], ea2, u2)


# speedup vs baseline: 1.2069x; 1.2069x over previous
import functools

import jax
import jax.numpy as jnp
from jax import lax
from jax.experimental import pallas as pl
from jax.experimental.pallas import tpu as pltpu
from jax.experimental.pallas import tpu_sc as plsc

N = 10000
E = 320000
D = 128
DE = 16
G = 16
DU = 32

NC = 2
NS = 16
NW = NC * NS
EW = E // NW
C = 80
K = 25
S = K * C
NSUP = EW // S
NP = 10240
RPS = NP // NS

BE = 40000
BN = 2048


def _eap_body(ea_ref, wbig_ref, be_ref, out_ref):
    out_ref[...] = (
        jnp.dot(ea_ref[...], wbig_ref[...], preferred_element_type=jnp.float32)
        + be_ref[...]
    )


def _proj_body(x_ref, batch_ref, u_ref, we_ref, asrc_ref, xd_ref):
    x = x_ref[...]
    xs = jnp.dot(x, we_ref[0:D, :], preferred_element_type=jnp.float32)
    xd = jnp.dot(x, we_ref[D:2 * D, :], preferred_element_type=jnp.float32)
    uweu = jnp.dot(u_ref[...], we_ref[2 * D + DE:2 * D + DE + DU, :],
                   preferred_element_type=jnp.float32)
    oh = (batch_ref[...] == lax.broadcasted_iota(jnp.int32, (N, G), 1)
          ).astype(jnp.float32)
    asrc_ref[...] = xs + jnp.dot(oh, uweu, preferred_element_type=jnp.float32)
    xd_ref[...] = xd


def _edge_sc_body(asrc_hbm, xd_hbm, eap_hbm, sidx_hbm, didx_hbm, zeros_hbm,
                  ea2_hbm, aggp_hbm,
                  sidx_v, didx_v, abuf, bbuf, obuf, agg_sh,
                  sem_i, sem_e, sem_g, sem_o, sem_s):
    c = lax.axis_index("c")
    s = lax.axis_index("s")
    wid = c * NS + s

    pltpu.sync_copy(zeros_hbm.at[pl.ds(s * RPS, RPS), :],
                    agg_sh.at[pl.ds(s * RPS, RPS), :])
    plsc.subcore_barrier()

    rbase0 = wid * (EW // C)

    def sup_body(i, carry):
        rbase = rbase0 + i * K
        ebase = (rbase0 + i * K) * C
        ci = pltpu.async_copy(sidx_hbm.at[pl.ds(rbase, K), :], sidx_v, sem_i)
        cj = pltpu.async_copy(didx_hbm.at[pl.ds(rbase, K), :], didx_v, sem_i)
        ce = pltpu.async_copy(eap_hbm.at[pl.ds(ebase // 8, S // 8), :],
                              obuf, sem_e)
        ci.wait()
        cj.wait()

        def gfire(j, carry2):
            pltpu.async_copy(asrc_hbm.at[sidx_v.at[j]],
                             abuf.at[pl.ds(j * C, C), :], sem_g)
            pltpu.async_copy(xd_hbm.at[didx_v.at[j]],
                             bbuf.at[pl.ds(j * C, C), :], sem_g)
            return carry2

        lax.fori_loop(0, K, gfire, 0)
        pltpu.make_async_copy(asrc_hbm.at[pl.ds(0, S), :], abuf, sem_g).wait()
        pltpu.make_async_copy(asrc_hbm.at[pl.ds(0, S), :], bbuf, sem_g).wait()
        ce.wait()

        def row(rr, _):
            for k in range(8):
                j = rr * 8 + k
                v = abuf[j, :] + bbuf[j, :] + obuf[rr, k * 16:(k + 1) * 16]
                abuf[j, :] = jnp.maximum(v, 0.0)
            return 0

        lax.fori_loop(0, S // 8, row, 0, unroll=2)
        co = pltpu.async_copy(abuf, ea2_hbm.at[pl.ds(ebase, S), :], sem_o)

        def sfire(j, carry2):
            pltpu.async_copy(abuf.at[pl.ds(j * C, C), :],
                             agg_sh.at[didx_v.at[j]], sem_s, add=True)
            return carry2

        lax.fori_loop(0, K, sfire, 0)
        pltpu.make_async_copy(asrc_hbm.at[pl.ds(0, S), :], abuf, sem_s).wait()
        co.wait()
        return carry

    lax.fori_loop(0, NSUP, sup_body, 0)

    plsc.subcore_barrier()
    pltpu.sync_copy(agg_sh.at[pl.ds(s * RPS, RPS), :],
                    abuf.at[pl.ds(0, RPS), :])

    def pack(rr, _):
        for k in range(8):
            obuf[rr, k * 16:(k + 1) * 16] = abuf[rr * 8 + k, :]
        return 0

    lax.fori_loop(0, RPS // 8, pack, 0, unroll=2)
    pltpu.sync_copy(obuf.at[pl.ds(0, RPS // 8), :],
                    aggp_hbm.at[c, pl.ds(s * (RPS // 8), RPS // 8), :])


@functools.cache
def _edge_sc():
    return pl.kernel(
        _edge_sc_body,
        out_type=[
            jax.ShapeDtypeStruct((E, DE), jnp.float32),
            jax.ShapeDtypeStruct((NC, NP * DE // 128, 128), jnp.float32),
        ],
        mesh=plsc.VectorSubcoreMesh(core_axis_name="c", subcore_axis_name="s",
                                    num_cores=NC, num_subcores=NS),
        compiler_params=pltpu.CompilerParams(use_tc_tiling_on_sc=False),
        scratch_types=[
            pltpu.VMEM((K, C), jnp.int32),
            pltpu.VMEM((K, C), jnp.int32),
            pltpu.VMEM((S, DE), jnp.float32),
            pltpu.VMEM((S, DE), jnp.float32),
            pltpu.VMEM((S // 8, 128), jnp.float32),
            pltpu.VMEM_SHARED((NP, DE), jnp.float32),
            pltpu.SemaphoreType.DMA,
            pltpu.SemaphoreType.DMA,
            pltpu.SemaphoreType.DMA,
            pltpu.SemaphoreType.DMA,
            pltpu.SemaphoreType.DMA,
        ],
    )


def _node_body(x_ref, aggp_ref, batch_ref, pp_ref, u_ref, wn_ref, bn_ref,
               wg_ref, bg_ref, x2_ref, u2_ref, accx_ref, accp_ref, accc_ref):
    i = pl.program_id(0)
    x = x_ref[...]
    agg = aggp_ref[...]
    oh = (batch_ref[...] == lax.broadcasted_iota(jnp.int32, (BN, G), 1)
          ).astype(jnp.float32)
    uwnu = jnp.dot(u_ref[...], wn_ref[D + DE:D + DE + DU, :],
                   preferred_element_type=jnp.float32)
    h = (jnp.dot(x, wn_ref[0:D, :], preferred_element_type=jnp.float32)
         + jnp.dot(agg, wn_ref[D:D + DE, :], preferred_element_type=jnp.float32)
         + jnp.dot(oh, uwnu, preferred_element_type=jnp.float32)
         + bn_ref[...])
    x2 = jnp.maximum(h, 0.0)
    x2_ref[...] = x2

    @pl.when(i == 0)
    def _():
        accx_ref[...] = jnp.zeros_like(accx_ref)
        accp_ref[...] = jnp.zeros_like(accp_ref)
        accc_ref[...] = jnp.zeros_like(accc_ref)

    dn = (((0,), (0,)), ((), ()))
    accx_ref[...] += lax.dot_general(oh, x2, dn,
                                     preferred_element_type=jnp.float32)
    accp_ref[...] += lax.dot_general(oh, pp_ref[...], dn,
                                     preferred_element_type=jnp.float32)
    accc_ref[...] += lax.dot_general(oh, jnp.ones((BN, D), jnp.float32), dn,
                                     preferred_element_type=jnp.float32)

    @pl.when(i == pl.num_programs(0) - 1)
    def _():
        cnt = jnp.maximum(accc_ref[...], 1.0)
        px = accx_ref[...] / cnt
        ppool = accp_ref[...] / cnt[:, 0:2]
        g = (jnp.dot(px, wg_ref[0:D, :], preferred_element_type=jnp.float32)
             + jnp.dot(ppool, wg_ref[D:D + 2, :],
                       preferred_element_type=jnp.float32)
             + jnp.dot(u_ref[...], wg_ref[D + 2:D + 2 + DU, :],
                       preferred_element_type=jnp.float32)
             + bg_ref[...])
        u2_ref[...] = jnp.maximum(g, 0.0)


def kernel(x, edge_index, edge_attr, u, batch, polar_pos, We, be, Wn, bn,
           Wg, bg):
    src = edge_index[0].astype(jnp.int32).reshape(E // C, C)
    dst = edge_index[1].astype(jnp.int32).reshape(E // C, C)
    batch2d = batch.astype(jnp.int32).reshape(N, 1)
    bn2 = bn.reshape(1, D)
    bg2 = bg.reshape(1, DU)
    zeros = jnp.zeros((NP, DE), jnp.float32)

    be128 = jnp.tile(be, 8).reshape(1, 128)

    a_src, xd = pl.pallas_call(
        _proj_body,
        in_specs=[
            pl.BlockSpec((N, D), lambda: (0, 0)),
            pl.BlockSpec((N, 1), lambda: (0, 0)),
            pl.BlockSpec((G, DU), lambda: (0, 0)),
            pl.BlockSpec((2 * D + DE + DU, DE), lambda: (0, 0)),
        ],
        out_specs=[
            pl.BlockSpec((N, DE), lambda: (0, 0)),
            pl.BlockSpec((N, DE), lambda: (0, 0)),
        ],
        out_shape=[
            jax.ShapeDtypeStruct((N, DE), jnp.float32),
            jax.ShapeDtypeStruct((N, DE), jnp.float32),
        ],
    )(x, batch2d, u, We)

    ea128 = edge_attr.reshape(E // 8, 128)
    wbig = jnp.kron(jnp.eye(8, dtype=jnp.float32), We[2 * D:2 * D + DE, :])

    eap = pl.pallas_call(
        _eap_body,
        grid=(E // BE,),
        in_specs=[
            pl.BlockSpec((BE // 8, 128), lambda i: (i, 0)),
            pl.BlockSpec((128, 128), lambda i: (0, 0)),
            pl.BlockSpec((1, 128), lambda i: (0, 0)),
        ],
        out_specs=pl.BlockSpec((BE // 8, 128), lambda i: (i, 0)),
        out_shape=jax.ShapeDtypeStruct((E // 8, 128), jnp.float32),
    )(ea128, wbig, be128)

    ea2, aggp = _edge_sc()(a_src, xd, eap, src, dst, zeros)
    agg2 = (aggp[0] + aggp[1]).reshape(NP, DE)

    xp = jnp.pad(x, ((0, NP - N), (0, 0)))
    batchp = jnp.pad(batch2d, ((0, NP - N), (0, 0)), constant_values=G)
    ppp = jnp.pad(polar_pos, ((0, NP - N), (0, 0)))

    x2p, u2 = pl.pallas_call(
        _node_body,
        grid=(NP // BN,),
        in_specs=[
            pl.BlockSpec((BN, D), lambda i: (i, 0)),
            pl.BlockSpec((BN, DE), lambda i: (i, 0)),
            pl.BlockSpec((BN, 1), lambda i: (i, 0)),
            pl.BlockSpec((BN, 2), lambda i: (i, 0)),
            pl.BlockSpec((G, DU), lambda i: (0, 0)),
            pl.BlockSpec((D + DE + DU, D), lambda i: (0, 0)),
            pl.BlockSpec((1, D), lambda i: (0, 0)),
            pl.BlockSpec((D + 2 + DU, DU), lambda i: (0, 0)),
            pl.BlockSpec((1, DU), lambda i: (0, 0)),
        ],
        out_specs=[
            pl.BlockSpec((BN, D), lambda i: (i, 0)),
            pl.BlockSpec((G, DU), lambda i: (0, 0)),
        ],
        out_shape=[
            jax.ShapeDtypeStruct((NP, D), jnp.float32),
            jax.ShapeDtypeStruct((G, DU), jnp.float32),
        ],
        scratch_shapes=[
            pltpu.VMEM((G, D), jnp.float32),
            pltpu.VMEM((G, 2), jnp.float32),
            pltpu.VMEM((G, D), jnp.float32),
        ],
    )(xp, agg2, batchp, ppp, u, Wn, bn2, Wg, bg2)

    return (x2p[:N], ea2, u2)
